# initial kernel scaffold (unmeasured)
import jax
import jax.numpy as jnp
from jax import lax
from jax.experimental import pallas as pl
from jax.experimental.pallas import tpu as pltpu

N_DEV = 8


def kernel(x, w_mat):
    m_per, k = x.shape
    _, n_per = w_mat.shape
    n_hops = N_DEV - 1

    def body(x_ref, w_ref, out_ref, xfull_ref, send_sems, recv_sems):
        my = lax.axis_index("i")
        left = (my - 1) % N_DEV
        right = (my + 1) % N_DEV

        barrier_sem = pltpu.get_barrier_semaphore()
        for nbr in [left, right]:
            pl.semaphore_signal(
                barrier_sem, inc=1,
                device_id=(nbr,), device_id_type=pl.DeviceIdType.MESH,
            )
        pl.semaphore_wait(barrier_sem, 2)

        xfull_ref[my] = x_ref[:, :]

        for h in range(n_hops):
            src_slot = (my - h) % N_DEV
            rdma = pltpu.make_async_remote_copy(
                src_ref=xfull_ref.at[src_slot],
                dst_ref=xfull_ref.at[src_slot],
                send_sem=send_sems.at[h],
                recv_sem=recv_sems.at[h],
                device_id=(right,),
                device_id_type=pl.DeviceIdType.MESH,
            )
            rdma.start()
            rdma.wait()

        xf = xfull_ref[:, :, :].reshape(N_DEV * m_per, k)
        y = jnp.dot(xf, w_ref[:, :], preferred_element_type=jnp.float32)
        out_ref[:, :] = y * (1.0 / (1.0 + jnp.exp(-y)))

    return pl.pallas_call(
        body,
        out_shape=jax.ShapeDtypeStruct((N_DEV * m_per, n_per), jnp.float32),
        in_specs=[
            pl.BlockSpec(memory_space=pltpu.VMEM),
            pl.BlockSpec(memory_space=pltpu.VMEM),
        ],
        out_specs=pl.BlockSpec(memory_space=pltpu.VMEM),
        scratch_shapes=[
            pltpu.VMEM((N_DEV, m_per, k), x.dtype),
            pltpu.SemaphoreType.DMA((n_hops,)),
            pltpu.SemaphoreType.DMA((n_hops,)),
        ],
        compiler_params=pltpu.CompilerParams(collective_id=0),
    )(x, w_mat)


# baseline (device time: 352159 ns/iter reference)
import jax
import jax.numpy as jnp
from jax import lax
from jax.experimental import pallas as pl
from jax.experimental.pallas import tpu as pltpu

N_DEV = 8


def kernel(x, w_mat):
    m_per, k = x.shape
    _, n_per = w_mat.shape
    n_hops = N_DEV - 1

    def body(x_ref, w_ref, out_ref, xfull_ref, wbf_ref, send_sems, recv_sems):
        my = lax.axis_index("i")
        left = (my - 1) % N_DEV
        right = (my + 1) % N_DEV

        barrier_sem = pltpu.get_barrier_semaphore()
        for nbr in [left, right]:
            pl.semaphore_signal(
                barrier_sem, inc=1,
                device_id=(nbr,), device_id_type=pl.DeviceIdType.MESH,
            )
        pl.semaphore_wait(barrier_sem, 2)

        wbf_ref[:, :] = w_ref[:, :].astype(jnp.bfloat16)
        xfull_ref[my] = x_ref[:, :].astype(jnp.bfloat16)

        for h in range(n_hops):
            src_slot = (my - h) % N_DEV
            rdma = pltpu.make_async_remote_copy(
                src_ref=xfull_ref.at[src_slot],
                dst_ref=xfull_ref.at[src_slot],
                send_sem=send_sems.at[h],
                recv_sem=recv_sems.at[h],
                device_id=(right,),
                device_id_type=pl.DeviceIdType.MESH,
            )
            rdma.start()
            rdma.wait()

        for s in range(N_DEV):
            y = jnp.dot(
                xfull_ref[s], wbf_ref[:, :], preferred_element_type=jnp.float32
            )
            out_ref[pl.ds(s * m_per, m_per), :] = y * (1.0 / (1.0 + jnp.exp(-y)))

    return pl.pallas_call(
        body,
        out_shape=jax.ShapeDtypeStruct((N_DEV * m_per, n_per), jnp.float32),
        in_specs=[
            pl.BlockSpec(memory_space=pltpu.VMEM),
            pl.BlockSpec(memory_space=pltpu.VMEM),
        ],
        out_specs=pl.BlockSpec(memory_space=pltpu.VMEM),
        scratch_shapes=[
            pltpu.VMEM((N_DEV, m_per, k), jnp.bfloat16),
            pltpu.VMEM((k, n_per), jnp.bfloat16),
            pltpu.SemaphoreType.DMA((n_hops,)),
            pltpu.SemaphoreType.DMA((n_hops,)),
        ],
        compiler_params=pltpu.CompilerParams(
            collective_id=0,
            vmem_limit_bytes=60 * 1024 * 1024,
        ),
    )(x, w_mat)


# device time: 189392 ns/iter; 1.8594x vs baseline; 1.8594x over previous
import jax
import jax.numpy as jnp
from jax import lax
from jax.experimental import pallas as pl
from jax.experimental.pallas import tpu as pltpu

N_DEV = 8


def kernel(x, w_mat):
    m_per, k = x.shape
    _, n_per = w_mat.shape
    n_hops = N_DEV - 1
    m_half = m_per // 2

    def body(x_ref, w_ref, out_ref, xg_ref, wbf_ref,
             send_r, recv_r, send_l, recv_l):
        my = lax.axis_index("i")
        left = (my - 1) % N_DEV
        right = (my + 1) % N_DEV

        barrier_sem = pltpu.get_barrier_semaphore()
        for nbr in [left, right]:
            pl.semaphore_signal(
                barrier_sem, inc=1,
                device_id=(nbr,), device_id_type=pl.DeviceIdType.MESH,
            )
        pl.semaphore_wait(barrier_sem, 2)

        xg_ref[my, 0] = x_ref[pl.ds(0, m_half), :].astype(jnp.bfloat16)
        xg_ref[my, 1] = x_ref[pl.ds(m_half, m_half), :].astype(jnp.bfloat16)

        def hop_sends(h):
            o_r = (my - h) % N_DEV
            o_l = (my + h) % N_DEV
            r = pltpu.make_async_remote_copy(
                src_ref=xg_ref.at[o_r, 0], dst_ref=xg_ref.at[o_r, 0],
                send_sem=send_r.at[h], recv_sem=recv_r.at[h],
                device_id=(right,), device_id_type=pl.DeviceIdType.MESH,
            )
            l = pltpu.make_async_remote_copy(
                src_ref=xg_ref.at[o_l, 1], dst_ref=xg_ref.at[o_l, 1],
                send_sem=send_l.at[h], recv_sem=recv_l.at[h],
                device_id=(left,), device_id_type=pl.DeviceIdType.MESH,
            )
            r.start()
            l.start()
            return r, l

        def gemm_half(origin, half):
            y = jnp.dot(
                xg_ref[origin, half], wbf_ref[:, :],
                preferred_element_type=jnp.float32,
            )
            row = origin * m_per + half * m_half
            out_ref[pl.ds(row, m_half), :] = y * (1.0 / (1.0 + jnp.exp(-y)))

        rdmas = [hop_sends(0)]
        wbf_ref[:, :] = w_ref[:, :].astype(jnp.bfloat16)
        gemm_half(my, 0)
        gemm_half(my, 1)

        for h in range(n_hops):
            r, l = rdmas[h]
            r.wait_recv()
            l.wait_recv()
            if h + 1 < n_hops:
                rdmas.append(hop_sends(h + 1))
            gemm_half((my - h - 1) % N_DEV, 0)
            gemm_half((my + h + 1) % N_DEV, 1)

        for r, l in rdmas:
            r.wait_send()
            l.wait_send()

    return pl.pallas_call(
        body,
        out_shape=jax.ShapeDtypeStruct((N_DEV * m_per, n_per), jnp.float32),
        in_specs=[
            pl.BlockSpec(memory_space=pltpu.VMEM),
            pl.BlockSpec(memory_space=pltpu.VMEM),
        ],
        out_specs=pl.BlockSpec(memory_space=pltpu.VMEM),
        scratch_shapes=[
            pltpu.VMEM((N_DEV, 2, m_half, k), jnp.bfloat16),
            pltpu.VMEM((k, n_per), jnp.bfloat16),
            pltpu.SemaphoreType.DMA((n_hops,)),
            pltpu.SemaphoreType.DMA((n_hops,)),
            pltpu.SemaphoreType.DMA((n_hops,)),
            pltpu.SemaphoreType.DMA((n_hops,)),
        ],
        compiler_params=pltpu.CompilerParams(
            collective_id=0,
            vmem_limit_bytes=60 * 1024 * 1024,
        ),
    )(x, w_mat)


# device time: 185716 ns/iter; 1.8962x vs baseline; 1.0198x over previous
import jax
import jax.numpy as jnp
from jax import lax
from jax.experimental import pallas as pl
from jax.experimental.pallas import tpu as pltpu

N_DEV = 8


def kernel(x, w_mat):
    m_per, k = x.shape
    _, n_per = w_mat.shape
    n_hops = N_DEV - 1
    m_half = m_per // 2

    def body(x_ref, w_ref, out_ref, xg_ref, wbf_ref,
             send_r, recv_r, send_l, recv_l):
        my = lax.axis_index("i")

        def pos_of(rr):
            rr = rr % N_DEV
            return jnp.where(rr < 4, rr, 11 - rr)

        rank = pos_of(my)
        left = pos_of(rank - 1)
        right = pos_of(rank + 1)

        barrier_sem = pltpu.get_barrier_semaphore()
        for nbr in [left, right]:
            pl.semaphore_signal(
                barrier_sem, inc=1,
                device_id=(nbr,), device_id_type=pl.DeviceIdType.MESH,
            )
        pl.semaphore_wait(barrier_sem, 2)

        xg_ref[my, 0] = x_ref[pl.ds(0, m_half), :].astype(jnp.bfloat16)
        xg_ref[my, 1] = x_ref[pl.ds(m_half, m_half), :].astype(jnp.bfloat16)

        def send_right(h):
            o_r = pos_of(rank - h)
            r = pltpu.make_async_remote_copy(
                src_ref=xg_ref.at[o_r, 0], dst_ref=xg_ref.at[o_r, 0],
                send_sem=send_r.at[h], recv_sem=recv_r.at[h],
                device_id=(right,), device_id_type=pl.DeviceIdType.MESH,
            )
            r.start()
            return r

        def send_left(h):
            o_l = pos_of(rank + h)
            l = pltpu.make_async_remote_copy(
                src_ref=xg_ref.at[o_l, 1], dst_ref=xg_ref.at[o_l, 1],
                send_sem=send_l.at[h], recv_sem=recv_l.at[h],
                device_id=(left,), device_id_type=pl.DeviceIdType.MESH,
            )
            l.start()
            return l

        def gemm_half(origin, half):
            y = jnp.dot(
                xg_ref[origin, half], wbf_ref[:, :],
                preferred_element_type=jnp.float32,
            )
            row = origin * m_per + half * m_half
            out_ref[pl.ds(row, m_half), :] = y * (1.0 / (1.0 + jnp.exp(-y)))

        rdmas = [(send_right(0), send_left(0))]
        wbf_ref[:, :] = w_ref[:, :].astype(jnp.bfloat16)
        gemm_half(my, 0)
        gemm_half(my, 1)

        for h in range(n_hops):
            r, l = rdmas[h]
            r.wait_recv()
            if h + 1 < n_hops:
                nr = send_right(h + 1)
            l.wait_recv()
            if h + 1 < n_hops:
                rdmas.append((nr, send_left(h + 1)))
            gemm_half(pos_of(rank - h - 1), 0)
            gemm_half(pos_of(rank + h + 1), 1)

        for r, l in rdmas:
            r.wait_send()
            l.wait_send()

    return pl.pallas_call(
        body,
        out_shape=jax.ShapeDtypeStruct((N_DEV * m_per, n_per), jnp.float32),
        in_specs=[
            pl.BlockSpec(memory_space=pltpu.VMEM),
            pl.BlockSpec(memory_space=pltpu.VMEM),
        ],
        out_specs=pl.BlockSpec(memory_space=pltpu.VMEM),
        scratch_shapes=[
            pltpu.VMEM((N_DEV, 2, m_half, k), jnp.bfloat16),
            pltpu.VMEM((k, n_per), jnp.bfloat16),
            pltpu.SemaphoreType.DMA((n_hops,)),
            pltpu.SemaphoreType.DMA((n_hops,)),
            pltpu.SemaphoreType.DMA((n_hops,)),
            pltpu.SemaphoreType.DMA((n_hops,)),
        ],
        compiler_params=pltpu.CompilerParams(
            collective_id=0,
            vmem_limit_bytes=60 * 1024 * 1024,
        ),
    )(x, w_mat)


# device time: 175903 ns/iter; 2.0020x vs baseline; 1.0558x over previous
import jax
import jax.numpy as jnp
from jax import lax
from jax.experimental import pallas as pl
from jax.experimental.pallas import tpu as pltpu

N_DEV = 8


def kernel(x, w_mat):
    m_per, k = x.shape
    _, n_per = w_mat.shape
    n_hops = N_DEV - 1
    m_half = m_per // 2

    def body(x_ref, w_ref, out_ref, xg_ref, wbf_ref,
             send_r, recv_r, send_l, recv_l):
        my = lax.axis_index("i")

        def pos_of(rr):
            rr = rr % N_DEV
            return jnp.where(rr < 4, rr, 11 - rr)

        rank = pos_of(my)
        left = pos_of(rank - 1)
        right = pos_of(rank + 1)

        barrier_sem = pltpu.get_barrier_semaphore()
        for nbr in [left, right]:
            pl.semaphore_signal(
                barrier_sem, inc=1,
                device_id=(nbr,), device_id_type=pl.DeviceIdType.MESH,
            )
        pl.semaphore_wait(barrier_sem, 2)

        xg_ref[my, 0] = x_ref[pl.ds(0, m_half), :].astype(jnp.bfloat16)
        xg_ref[my, 1] = x_ref[pl.ds(m_half, m_half), :].astype(jnp.bfloat16)

        m_q = m_half // 2

        def send_right(h, q):
            o_r = pos_of(rank - h)
            r = pltpu.make_async_remote_copy(
                src_ref=xg_ref.at[o_r, 0, pl.ds(q * m_q, m_q)],
                dst_ref=xg_ref.at[o_r, 0, pl.ds(q * m_q, m_q)],
                send_sem=send_r.at[h, q], recv_sem=recv_r.at[h, q],
                device_id=(right,), device_id_type=pl.DeviceIdType.MESH,
            )
            r.start()
            return r

        def send_left(h, q):
            o_l = pos_of(rank + h)
            l = pltpu.make_async_remote_copy(
                src_ref=xg_ref.at[o_l, 1, pl.ds(q * m_q, m_q)],
                dst_ref=xg_ref.at[o_l, 1, pl.ds(q * m_q, m_q)],
                send_sem=send_l.at[h, q], recv_sem=recv_l.at[h, q],
                device_id=(left,), device_id_type=pl.DeviceIdType.MESH,
            )
            l.start()
            return l

        def gemm_half(origin, half):
            y = jnp.dot(
                xg_ref[origin, half], wbf_ref[:, :],
                preferred_element_type=jnp.float32,
            )
            row = origin * m_per + half * m_half
            out_ref[pl.ds(row, m_half), :] = y * (1.0 / (1.0 + jnp.exp(-y)))

        rdmas = [(send_right(0, 0), send_right(0, 1),
                  send_left(0, 0), send_left(0, 1))]
        wbf_ref[:, :] = w_ref[:, :].astype(jnp.bfloat16)
        gemm_half(my, 0)
        gemm_half(my, 1)

        for h in range(n_hops):
            rq0, rq1, lq0, lq1 = rdmas[h]
            rq0.wait_recv()
            if h + 1 < n_hops:
                nr0 = send_right(h + 1, 0)
            lq0.wait_recv()
            if h + 1 < n_hops:
                nl0 = send_left(h + 1, 0)
            rq1.wait_recv()
            if h + 1 < n_hops:
                nr1 = send_right(h + 1, 1)
            lq1.wait_recv()
            if h + 1 < n_hops:
                rdmas.append((nr0, nr1, nl0, send_left(h + 1, 1)))
            gemm_half(pos_of(rank - h - 1), 0)
            gemm_half(pos_of(rank + h + 1), 1)

        for group in rdmas:
            for d in group:
                d.wait_send()

    return pl.pallas_call(
        body,
        out_shape=jax.ShapeDtypeStruct((N_DEV * m_per, n_per), jnp.float32),
        in_specs=[
            pl.BlockSpec(memory_space=pltpu.VMEM),
            pl.BlockSpec(memory_space=pltpu.VMEM),
        ],
        out_specs=pl.BlockSpec(memory_space=pltpu.VMEM),
        scratch_shapes=[
            pltpu.VMEM((N_DEV, 2, m_half, k), jnp.bfloat16),
            pltpu.VMEM((k, n_per), jnp.bfloat16),
            pltpu.SemaphoreType.DMA((n_hops, 2)),
            pltpu.SemaphoreType.DMA((n_hops, 2)),
            pltpu.SemaphoreType.DMA((n_hops, 2)),
            pltpu.SemaphoreType.DMA((n_hops, 2)),
        ],
        compiler_params=pltpu.CompilerParams(
            collective_id=0,
            vmem_limit_bytes=60 * 1024 * 1024,
        ),
    )(x, w_mat)


# device time: 126825 ns/iter; 2.7767x vs baseline; 1.3870x over previous
import jax
import jax.numpy as jnp
from jax import lax
from jax.experimental import pallas as pl
from jax.experimental.pallas import tpu as pltpu

N_DEV = 8

_PARTS = ((0, 176), (176, 176), (352, 160))
_DIMS = ((1, 3, 4), (3, 4, 1), (4, 1, 3))


def kernel(x, w_mat):
    m_per, k = x.shape
    _, n_per = w_mat.shape

    def body(x_ref, w_ref, out_ref, xg_ref, wbf_ref, send_sems, recv_sems):
        my = lax.axis_index("i")

        barrier_sem = pltpu.get_barrier_semaphore()
        for mask in (1, 3, 4):
            pl.semaphore_signal(
                barrier_sem, inc=1,
                device_id=(my ^ mask,), device_id_type=pl.DeviceIdType.MESH,
            )
        pl.semaphore_wait(barrier_sem, 3)

        xg_ref[my] = x_ref[:, :].astype(jnp.bfloat16)

        _DST = ((0,), (1,), (1,), (2,), (2,), (2,), (2,))

        def send_origin(s, i):
            d1, d2, _ = _DIMS[s]
            return (my, my, my ^ d1, my, my ^ d1, my ^ d2, my ^ d2 ^ d1)[i]

        def recv_origin(s, i):
            d1, d2, d3 = _DIMS[s]
            p = (my ^ d1, my ^ d2, my ^ d2, my ^ d3, my ^ d3, my ^ d3,
                 my ^ d3)[i]
            return (p, p, p ^ d1, p, p ^ d1, p ^ d2, p ^ d2 ^ d1)[i]

        def send(s, i):
            off, ln = _PARTS[s]
            o = send_origin(s, i)
            dst = my ^ _DIMS[s][_DST[i][0]]
            d = pltpu.make_async_remote_copy(
                src_ref=xg_ref.at[o, pl.ds(off, ln)],
                dst_ref=xg_ref.at[o, pl.ds(off, ln)],
                send_sem=send_sems.at[s, i], recv_sem=recv_sems.at[s, i],
                device_id=(dst,), device_id_type=pl.DeviceIdType.MESH,
            )
            d.start()
            return d

        def recv_wait_gemm(s, i):
            off, ln = _PARTS[s]
            o = recv_origin(s, i)
            dummy = pltpu.make_async_remote_copy(
                src_ref=xg_ref.at[o, pl.ds(off, ln)],
                dst_ref=xg_ref.at[o, pl.ds(off, ln)],
                send_sem=send_sems.at[s, i], recv_sem=recv_sems.at[s, i],
                device_id=(my,), device_id_type=pl.DeviceIdType.MESH,
            )
            dummy.wait_recv()
            y = jnp.dot(
                xg_ref[o, pl.ds(off, ln)], wbf_ref[:, :],
                preferred_element_type=jnp.float32,
            )
            row = o * m_per + off
            out_ref[pl.ds(row, ln), :] = y * (1.0 / (1.0 + jnp.exp(-y)))

        started = []

        def issue(s, i):
            started.append(send(s, i))

        for s in range(3):
            issue(s, 0)
        for s in range(3):
            issue(s, 1)
        for s in range(3):
            issue(s, 3)

        wbf_ref[:, :] = w_ref[:, :].astype(jnp.bfloat16)
        y = jnp.dot(xg_ref[my], wbf_ref[:, :],
                    preferred_element_type=jnp.float32)
        out_ref[pl.ds(my * m_per, m_per), :] = y * (1.0 / (1.0 + jnp.exp(-y)))

        for s in range(3):
            recv_wait_gemm(s, 0)
            issue(s, 2)
            issue(s, 4)
        for s in range(3):
            recv_wait_gemm(s, 1)
            issue(s, 5)
            recv_wait_gemm(s, 2)
            issue(s, 6)
        for i in (3, 4, 5, 6):
            for s in range(3):
                recv_wait_gemm(s, i)

        for d in started:
            d.wait_send()

    return pl.pallas_call(
        body,
        out_shape=jax.ShapeDtypeStruct((N_DEV * m_per, n_per), jnp.float32),
        in_specs=[
            pl.BlockSpec(memory_space=pltpu.VMEM),
            pl.BlockSpec(memory_space=pltpu.VMEM),
        ],
        out_specs=pl.BlockSpec(memory_space=pltpu.VMEM),
        scratch_shapes=[
            pltpu.VMEM((N_DEV, m_per, k), jnp.bfloat16),
            pltpu.VMEM((k, n_per), jnp.bfloat16),
            pltpu.SemaphoreType.DMA((3, 7)),
            pltpu.SemaphoreType.DMA((3, 7)),
        ],
        compiler_params=pltpu.CompilerParams(
            collective_id=0,
            vmem_limit_bytes=60 * 1024 * 1024,
        ),
    )(x, w_mat)


# device time: 126623 ns/iter; 2.7812x vs baseline; 1.0016x over previous
import jax
import jax.numpy as jnp
from jax import lax
from jax.experimental import pallas as pl
from jax.experimental.pallas import tpu as pltpu

N_DEV = 8

_PARTS = ((0, 176), (176, 176), (352, 160))
_DIMS = ((1, 3, 4), (3, 4, 1), (4, 1, 3))


def kernel(x, w_mat):
    m_per, k = x.shape
    _, n_per = w_mat.shape

    def body(x_ref, w_ref, out_ref, xg_ref, wbf_ref, send_sems, recv_sems):
        my = lax.axis_index("i")

        barrier_sem = pltpu.get_barrier_semaphore()
        for mask in (1, 3, 4):
            pl.semaphore_signal(
                barrier_sem, inc=1,
                device_id=(my ^ mask,), device_id_type=pl.DeviceIdType.MESH,
            )
        pl.semaphore_wait(barrier_sem, 3)


        _DST = ((0,), (1,), (1,), (2,), (2,), (2,), (2,))

        def send_origin(s, i):
            d1, d2, _ = _DIMS[s]
            return (my, my, my ^ d1, my, my ^ d1, my ^ d2, my ^ d2 ^ d1)[i]

        def recv_origin(s, i):
            d1, d2, d3 = _DIMS[s]
            p = (my ^ d1, my ^ d2, my ^ d2, my ^ d3, my ^ d3, my ^ d3,
                 my ^ d3)[i]
            return (p, p, p ^ d1, p, p ^ d1, p ^ d2, p ^ d2 ^ d1)[i]

        def send(s, i):
            off, ln = _PARTS[s]
            o = send_origin(s, i)
            dst = my ^ _DIMS[s][_DST[i][0]]
            d = pltpu.make_async_remote_copy(
                src_ref=xg_ref.at[o, pl.ds(off, ln)],
                dst_ref=xg_ref.at[o, pl.ds(off, ln)],
                send_sem=send_sems.at[s, i], recv_sem=recv_sems.at[s, i],
                device_id=(dst,), device_id_type=pl.DeviceIdType.MESH,
            )
            d.start()
            return d

        def recv_wait(s, i):
            off, ln = _PARTS[s]
            o = recv_origin(s, i)
            dummy = pltpu.make_async_remote_copy(
                src_ref=xg_ref.at[o, pl.ds(off, ln)],
                dst_ref=xg_ref.at[o, pl.ds(off, ln)],
                send_sem=send_sems.at[s, i], recv_sem=recv_sems.at[s, i],
                device_id=(my,), device_id_type=pl.DeviceIdType.MESH,
            )
            dummy.wait_recv()

        def gemm_unit(s, i):
            off, ln = _PARTS[s]
            o = recv_origin(s, i)
            y = jnp.dot(
                xg_ref[o, pl.ds(off, ln)], wbf_ref[:, :],
                preferred_element_type=jnp.float32,
            )
            row = o * m_per + off
            out_ref[pl.ds(row, ln), :] = y * (1.0 / (1.0 + jnp.exp(-y)))

        started = []

        def issue(s, i):
            started.append(send(s, i))

        for s in range(3):
            off, ln = _PARTS[s]
            xg_ref[my, pl.ds(off, ln)] = (
                x_ref[pl.ds(off, ln), :].astype(jnp.bfloat16)
            )
            issue(s, 0)
        for s in range(3):
            issue(s, 1)
        for s in range(3):
            issue(s, 3)

        wbf_ref[:, :] = w_ref[:, :].astype(jnp.bfloat16)
        y = jnp.dot(xg_ref[my], wbf_ref[:, :],
                    preferred_element_type=jnp.float32)
        out_ref[pl.ds(my * m_per, m_per), :] = y * (1.0 / (1.0 + jnp.exp(-y)))

        for s in range(3):
            recv_wait(s, 0)
            issue(s, 2)
            issue(s, 4)
            gemm_unit(s, 0)
        for s in range(3):
            recv_wait(s, 1)
            issue(s, 5)
            recv_wait(s, 2)
            issue(s, 6)
            gemm_unit(s, 1)
            gemm_unit(s, 2)
        for i in (3, 4, 5, 6):
            for s in range(3):
                recv_wait(s, i)
                gemm_unit(s, i)

        for d in started:
            d.wait_send()

    return pl.pallas_call(
        body,
        out_shape=jax.ShapeDtypeStruct((N_DEV * m_per, n_per), jnp.float32),
        in_specs=[
            pl.BlockSpec(memory_space=pltpu.VMEM),
            pl.BlockSpec(memory_space=pltpu.VMEM),
        ],
        out_specs=pl.BlockSpec(memory_space=pltpu.VMEM),
        scratch_shapes=[
            pltpu.VMEM((N_DEV, m_per, k), jnp.bfloat16),
            pltpu.VMEM((k, n_per), jnp.bfloat16),
            pltpu.SemaphoreType.DMA((3, 7)),
            pltpu.SemaphoreType.DMA((3, 7)),
        ],
        compiler_params=pltpu.CompilerParams(
            collective_id=0,
            vmem_limit_bytes=60 * 1024 * 1024,
        ),
    )(x, w_mat)


# device time: 91606 ns/iter; 3.8443x vs baseline; 1.3823x over previous
import jax
import jax.numpy as jnp
from jax import lax
from jax.experimental import pallas as pl
from jax.experimental.pallas import tpu as pltpu

N_DEV = 8

_DIMS = ((1, 3, 4), (4, 3, 1))


def kernel(x, w_mat):
    m_per, k = x.shape
    _, n_per = w_mat.shape
    n_half = n_per // 2

    def body(x_ref, w_ref, out_ref, xbf_ref, wg_ref, ybuf_ref, yin_ref,
             wsend_sems, wrecv_sems, ysend_sems, yrecv_sems):
        my = lax.axis_index("i")

        barrier_sem = pltpu.get_barrier_semaphore()
        for mask in (1, 3, 4):
            pl.semaphore_signal(
                barrier_sem, inc=1,
                device_id=(my ^ mask,), device_id_type=pl.DeviceIdType.MESH,
            )
        pl.semaphore_wait(barrier_sem, 3)

        _DSTDIM = (0, 1, 1, 2, 2, 2, 2)

        def send_origin(s, i):
            d1, d2, _ = _DIMS[s]
            return (my, my, my ^ d1, my, my ^ d1, my ^ d2, my ^ d2 ^ d1)[i]

        def recv_origin(s, i):
            d1, d2, d3 = _DIMS[s]
            p = (my ^ d1, my ^ d2, my ^ d2, my ^ d3, my ^ d3, my ^ d3,
                 my ^ d3)[i]
            return (p, p, p ^ d1, p, p ^ d1, p ^ d2, p ^ d2 ^ d1)[i]

        def wslot(o, s):
            return wg_ref.at[o, :, pl.ds(s * n_half, n_half)]

        def send(s, i):
            o = send_origin(s, i)
            dst = my ^ _DIMS[s][_DSTDIM[i]]
            d = pltpu.make_async_remote_copy(
                src_ref=wslot(o, s), dst_ref=wslot(o, s),
                send_sem=wsend_sems.at[s, i], recv_sem=wrecv_sems.at[s, i],
                device_id=(dst,), device_id_type=pl.DeviceIdType.MESH,
            )
            d.start()
            return d

        def recv_wait(s, i):
            o = recv_origin(s, i)
            dummy = pltpu.make_async_remote_copy(
                src_ref=wslot(o, s), dst_ref=wslot(o, s),
                send_sem=wsend_sems.at[s, i], recv_sem=wrecv_sems.at[s, i],
                device_id=(my,), device_id_type=pl.DeviceIdType.MESH,
            )
            dummy.wait_recv()

        def gemm_send(s, i):
            o = recv_origin(s, i)
            y = jnp.dot(
                xbf_ref[:, :], wg_ref[o, :, pl.ds(s * n_half, n_half)],
                preferred_element_type=jnp.float32,
            )
            ybuf_ref[o, :, pl.ds(s * n_half, n_half)] = (
                (y * (1.0 / (1.0 + jnp.exp(-y)))).astype(jnp.bfloat16)
            )
            d = pltpu.make_async_remote_copy(
                src_ref=ybuf_ref.at[o, :, pl.ds(s * n_half, n_half)],
                dst_ref=yin_ref.at[my, :, pl.ds(s * n_half, n_half)],
                send_sem=ysend_sems.at[o, s], recv_sem=yrecv_sems.at[my, s],
                device_id=(o,), device_id_type=pl.DeviceIdType.MESH,
            )
            d.start()
            return d

        wstarted = []
        ystarted = []

        def issue(s, i):
            wstarted.append(send(s, i))

        for s in range(2):
            wg_ref[my, :, pl.ds(s * n_half, n_half)] = (
                w_ref[:, pl.ds(s * n_half, n_half)].astype(jnp.bfloat16)
            )
            issue(s, 0)
        for s in range(2):
            issue(s, 1)
        for s in range(2):
            issue(s, 3)

        xbf_ref[:, :] = x_ref[:, :].astype(jnp.bfloat16)
        y = jnp.dot(xbf_ref[:, :], wg_ref[my],
                    preferred_element_type=jnp.float32)
        out_ref[pl.ds(my * m_per, m_per), :] = y * (1.0 / (1.0 + jnp.exp(-y)))

        for s in range(2):
            recv_wait(s, 0)
            issue(s, 2)
            issue(s, 4)
            ystarted.append(gemm_send(s, 0))
        for s in range(2):
            recv_wait(s, 1)
            issue(s, 5)
            recv_wait(s, 2)
            issue(s, 6)
            ystarted.append(gemm_send(s, 1))
            ystarted.append(gemm_send(s, 2))
        for i in (3, 4, 5, 6):
            for s in range(2):
                recv_wait(s, i)
                ystarted.append(gemm_send(s, i))

        for g in range(1, N_DEV):
            e = my ^ g
            for s in range(2):
                dummy = pltpu.make_async_remote_copy(
                    src_ref=ybuf_ref.at[e, :, pl.ds(s * n_half, n_half)],
                    dst_ref=yin_ref.at[e, :, pl.ds(s * n_half, n_half)],
                    send_sem=ysend_sems.at[e, s], recv_sem=yrecv_sems.at[e, s],
                    device_id=(my,), device_id_type=pl.DeviceIdType.MESH,
                )
                dummy.wait_recv()
            out_ref[pl.ds(e * m_per, m_per), :] = (
                yin_ref[e].astype(jnp.float32)
            )

        for d in wstarted:
            d.wait_send()
        for d in ystarted:
            d.wait_send()

    return pl.pallas_call(
        body,
        out_shape=jax.ShapeDtypeStruct((N_DEV * m_per, n_per), jnp.float32),
        in_specs=[
            pl.BlockSpec(memory_space=pltpu.VMEM),
            pl.BlockSpec(memory_space=pltpu.VMEM),
        ],
        out_specs=pl.BlockSpec(memory_space=pltpu.VMEM),
        scratch_shapes=[
            pltpu.VMEM((m_per, k), jnp.bfloat16),
            pltpu.VMEM((N_DEV, k, n_per), jnp.bfloat16),
            pltpu.VMEM((N_DEV, m_per, n_per), jnp.bfloat16),
            pltpu.VMEM((N_DEV, m_per, n_per), jnp.bfloat16),
            pltpu.SemaphoreType.DMA((2, 7)),
            pltpu.SemaphoreType.DMA((2, 7)),
            pltpu.SemaphoreType.DMA((N_DEV, 2)),
            pltpu.SemaphoreType.DMA((N_DEV, 2)),
        ],
        compiler_params=pltpu.CompilerParams(
            collective_id=0,
            vmem_limit_bytes=60 * 1024 * 1024,
        ),
    )(x, w_mat)
